# baseline (speedup 1.0000x reference)
import functools

import jax
import jax.numpy as jnp
from jax import lax
from jax.experimental import pallas as pl
from jax.experimental.pallas import tpu as pltpu
from jax.experimental.pallas import tpu_sc as plsc

K = 16
D = 16
BQ = 256
M_PAD = 8192
N_PAD = 10240
BQ2 = 512
NW = 32
B_PAD = 163840


def _leaky(x):
    return jnp.where(x >= 0, x, 0.2 * x)


def _atom_mlp_body(t_ref, w1_ref, b1_ref, w2_ref, b2_ref, w3_ref, b3_ref,
                   out_ref):
    x = t_ref[...]
    x = _leaky(jnp.dot(x, w1_ref[...],
                       preferred_element_type=jnp.float32) + b1_ref[...])
    x = _leaky(jnp.dot(x, w2_ref[...],
                       preferred_element_type=jnp.float32) + b2_ref[...])
    x = _leaky(jnp.dot(x, w3_ref[...],
                       preferred_element_type=jnp.float32) + b3_ref[...])
    out_ref[...] = x


def _atom_mlp(types_p, Wt1, bt1, Wt2, bt2, Wt3, bt3):
    return pl.pallas_call(
        _atom_mlp_body,
        out_shape=jax.ShapeDtypeStruct((M_PAD, D), jnp.float32),
    )(types_p, Wt1, bt1.reshape(1, D), Wt2, bt2.reshape(1, D),
      Wt3, bt3.reshape(1, D))


_NVJ = M_PAD // 128
_BIG = 3.3e38
_BIGI = 1 << 30


_DEPTH = 5


def _topk_body(xaug_ref, aaug_ref, idx_ref, dist_ref, w_ref):
    lane = lax.broadcasted_iota(jnp.int32, (1, 128), 1)
    w_ref[...] = jnp.dot(xaug_ref[...], aaug_ref[...],
                         preferred_element_type=jnp.float32)

    sv = [jnp.full((BQ, 128), _BIG, jnp.float32) for _ in range(_DEPTH)]
    si = [jnp.full((BQ, 128), _BIGI, jnp.int32) for _ in range(_DEPTH)]
    for vj in range(_NVJ):
        xv = w_ref[:, 128 * vj:128 * (vj + 1)]
        absv = lane + 128 * vj
        c = [xv < s for s in sv]
        for j in range(_DEPTH - 1, 0, -1):
            ins = jnp.where(c[j], xv, sv[j])
            sv[j] = jnp.where(c[j - 1], sv[j - 1], ins)
            insi = jnp.where(c[j], absv, si[j])
            si[j] = jnp.where(c[j - 1], si[j - 1], insi)
        sv[0] = jnp.where(c[0], xv, sv[0])
        si[0] = jnp.where(c[0], absv, si[0])

    vals, idxs = [], []
    fq = jnp.zeros((BQ, 1), jnp.bool_)
    for k in range(K):
        mv = jnp.min(sv[0], axis=1, keepdims=True)
        am = jnp.min(jnp.where(sv[0] <= mv, si[0], _BIGI), axis=1,
                     keepdims=True)
        vals.append(mv)
        idxs.append(am)
        m = lane == (am & 127)
        fq = fq | jnp.any(m & (sv[1] >= _BIG), axis=1, keepdims=True)
        for j in range(_DEPTH - 1):
            sv[j] = jnp.where(m, sv[j + 1], sv[j])
            si[j] = jnp.where(m, si[j + 1], si[j])
        sv[-1] = jnp.where(m, _BIG, sv[-1])
        si[-1] = jnp.where(m, _BIGI, si[-1])
    idx_ref[...] = jnp.concatenate(idxs, axis=1)
    dist_ref[...] = jnp.concatenate(vals, axis=1)

    @pl.when(jnp.any(fq))
    def _():
        ii = lax.broadcasted_iota(jnp.int32, (BQ, M_PAD), 1)
        vals2, idxs2 = [], []
        for k in range(K):
            w = w_ref[...]
            mv = jnp.min(w, axis=1, keepdims=True)
            am = jnp.min(jnp.where(w <= mv, ii, _BIGI), axis=1,
                         keepdims=True)
            vals2.append(mv)
            idxs2.append(am)
            w_ref[...] = jnp.where(ii == am, _BIG, w)
        idx_ref[...] = jnp.where(fq, jnp.concatenate(idxs2, axis=1),
                                 idx_ref[...])
        dist_ref[...] = jnp.where(fq, jnp.concatenate(vals2, axis=1),
                                  dist_ref[...])


def _topk(xyz_p, axyz_p):
    return pl.pallas_call(
        _topk_body,
        grid=(N_PAD // BQ,),
        in_specs=[
            pl.BlockSpec((BQ, 8), lambda i: (i, 0)),
            pl.BlockSpec((8, M_PAD), lambda i: (0, 0)),
        ],
        out_specs=[
            pl.BlockSpec((BQ, K), lambda i: (i, 0)),
            pl.BlockSpec((BQ, K), lambda i: (i, 0)),
        ],
        out_shape=[
            jax.ShapeDtypeStruct((N_PAD, K), jnp.int32),
            jax.ShapeDtypeStruct((N_PAD, K), jnp.float32),
        ],
        scratch_shapes=[pltpu.VMEM((BQ, M_PAD), jnp.float32)],
        compiler_params=pltpu.CompilerParams(
            dimension_semantics=("arbitrary",)),
    )(xyz_p, axyz_p)


def _sc_gather_body(table_hbm, idx_hbm, out_hbm, idx_v, rows_v, sem):
    wid = lax.axis_index("s") * 2 + lax.axis_index("c")
    rows_per_w = B_PAD // NW
    base = wid * rows_per_w
    pltpu.sync_copy(idx_hbm.at[pl.ds(base, rows_per_w)], idx_v)
    pltpu.async_copy(table_hbm.at[idx_v], rows_v, sem).wait()
    pltpu.sync_copy(rows_v, out_hbm.at[pl.ds(base, rows_per_w)])


def _sc_gather(table, flat_idx_p):
    rows_per_w = B_PAD // NW
    mesh = plsc.VectorSubcoreMesh(core_axis_name="c", subcore_axis_name="s")
    f = pl.kernel(
        _sc_gather_body,
        out_type=jax.ShapeDtypeStruct((B_PAD, D), jnp.float32),
        mesh=mesh,
        scratch_types=[
            pltpu.VMEM((rows_per_w,), jnp.int32),
            pltpu.VMEM((rows_per_w, D), jnp.float32),
            pltpu.SemaphoreType.DMA,
        ],
        compiler_params=pltpu.CompilerParams(use_tc_tiling_on_sc=False),
    )
    return f(table, flat_idx_p)


def _c1_body(n_valid, g_ref, d_ref, w1bd_ref, rw_ref, b1t_ref,
             h1_ref, s_ref):
    i = pl.program_id(0)
    x = jnp.dot(g_ref[...], w1bd_ref[...], preferred_element_type=jnp.float32)
    inv = 1.0 / d_ref[...]
    x = x + jnp.dot(inv, rw_ref[...], preferred_element_type=jnp.float32)
    h = _leaky(x + b1t_ref[...])
    h1_ref[...] = h
    rows = i * BQ2 + lax.broadcasted_iota(jnp.int32, (BQ2, 1), 0)
    msk = rows < n_valid
    hm = jnp.where(msk, h, 0.0)
    hm2 = jnp.where(msk, h * h, 0.0)

    @pl.when(i == 0)
    def _():
        s_ref[...] = jnp.zeros_like(s_ref)

    s_ref[0:1, :] += jnp.sum(hm, axis=0, keepdims=True)
    s_ref[1:2, :] += jnp.sum(hm2, axis=0, keepdims=True)


def _bn_consts(s_ref, n_total, gt_ref, bt_ref):
    s = s_ref[...]
    tot = None
    tot2 = None
    for k in range(K):
        sl = s[0:1, 16 * k:16 * (k + 1)]
        sl2 = s[1:2, 16 * k:16 * (k + 1)]
        tot = sl if tot is None else tot + sl
        tot2 = sl2 if tot2 is None else tot2 + sl2
    mean = tot / n_total
    var = tot2 / n_total - mean * mean
    den = jnp.sqrt(var + 1e-5)
    mt = jnp.concatenate([mean] * K, axis=1)
    dent = jnp.concatenate([den] * K, axis=1)
    return mt, dent, gt_ref[...], bt_ref[...]


def _c2_body(n_valid, h1_ref, s1_ref, g1t_ref, be1t_ref, w2bd_ref, b2t_ref,
             h2_ref, fx1_ref, s_ref):
    i = pl.program_id(0)
    mt, dent, gt, bt = _bn_consts(s1_ref, jnp.float32(n_valid * K),
                                  g1t_ref, be1t_ref)
    u = (h1_ref[...] - mt) / dent * gt + bt
    fx1 = None
    for k in range(K):
        sl = u[:, 16 * k:16 * (k + 1)]
        fx1 = sl if fx1 is None else fx1 + sl
    fx1_ref[...] = fx1
    h = _leaky(jnp.dot(u, w2bd_ref[...],
                       preferred_element_type=jnp.float32) + b2t_ref[...])
    h2_ref[...] = h
    rows = i * BQ2 + lax.broadcasted_iota(jnp.int32, (BQ2, 1), 0)
    msk = rows < n_valid
    hm = jnp.where(msk, h, 0.0)
    hm2 = jnp.where(msk, h * h, 0.0)

    @pl.when(i == 0)
    def _():
        s_ref[...] = jnp.zeros_like(s_ref)

    s_ref[0:1, :] += jnp.sum(hm, axis=0, keepdims=True)
    s_ref[1:2, :] += jnp.sum(hm2, axis=0, keepdims=True)


def _c3_body(n_valid, h2_ref, s2_ref, g2t_ref, be2t_ref, fx1_ref,
             w3a_ref, w3b_ref, b3_ref, out_ref):
    mt, dent, gt, bt = _bn_consts(s2_ref, jnp.float32(n_valid * K),
                                  g2t_ref, be2t_ref)
    v = (h2_ref[...] - mt) / dent * gt + bt
    fx2 = None
    for k in range(K):
        sl = v[:, 16 * k:16 * (k + 1)]
        fx2 = sl if fx2 is None else fx2 + sl
    out_ref[...] = (jnp.dot(fx1_ref[...], w3a_ref[...],
                            preferred_element_type=jnp.float32)
                    + jnp.dot(fx2, w3b_ref[...],
                              preferred_element_type=jnp.float32)
                    + b3_ref[...])


def _whole(shape):
    return pl.BlockSpec(shape, lambda i: tuple(0 for _ in shape))


def _rowblk(w):
    return pl.BlockSpec((BQ2, w), lambda i: (i, 0))


def _c1(n_valid, g2, d2p, w1bd, rw, b1t):
    return pl.pallas_call(
        functools.partial(_c1_body, n_valid),
        grid=(N_PAD // BQ2,),
        in_specs=[_rowblk(K * D), _rowblk(K), _whole((K * D, K * D)),
                  _whole((K, K * D)), _whole((1, K * D))],
        out_specs=[_rowblk(K * D), _whole((8, K * D))],
        out_shape=[
            jax.ShapeDtypeStruct((N_PAD, K * D), jnp.float32),
            jax.ShapeDtypeStruct((8, K * D), jnp.float32),
        ],
        compiler_params=pltpu.CompilerParams(
            dimension_semantics=("arbitrary",)),
    )(g2, d2p, w1bd, rw, b1t)


def _c2(n_valid, h1, s1, g1t, be1t, w2bd, b2t):
    return pl.pallas_call(
        functools.partial(_c2_body, n_valid),
        grid=(N_PAD // BQ2,),
        in_specs=[_rowblk(K * D), _whole((8, K * D)), _whole((1, K * D)),
                  _whole((1, K * D)), _whole((K * D, K * D)),
                  _whole((1, K * D))],
        out_specs=[_rowblk(K * D), _rowblk(D), _whole((8, K * D))],
        out_shape=[
            jax.ShapeDtypeStruct((N_PAD, K * D), jnp.float32),
            jax.ShapeDtypeStruct((N_PAD, D), jnp.float32),
            jax.ShapeDtypeStruct((8, K * D), jnp.float32),
        ],
        compiler_params=pltpu.CompilerParams(
            dimension_semantics=("arbitrary",)),
    )(h1, s1, g1t, be1t, w2bd, b2t)


def _c3(n_valid, h2, s2, g2t, be2t, fx1, w3a, w3b, b3):
    return pl.pallas_call(
        functools.partial(_c3_body, n_valid),
        grid=(N_PAD // BQ2,),
        in_specs=[_rowblk(K * D), _whole((8, K * D)), _whole((1, K * D)),
                  _whole((1, K * D)), _rowblk(D), _whole((D, D)),
                  _whole((D, D)), _whole((1, D))],
        out_specs=_rowblk(D),
        out_shape=jax.ShapeDtypeStruct((N_PAD, D), jnp.float32),
        compiler_params=pltpu.CompilerParams(
            dimension_semantics=("arbitrary",)),
    )(h2, s2, g2t, be2t, fx1, w3a, w3b, b3)


def kernel(xyz, atom_xyz, atom_types, Wt1, bt1, Wt2, bt2, Wt3, bt3,
           Wc1, bc1, Wc2, bc2, Wc3, bc3, g1, beta1, g2, beta2):
    N = xyz.shape[0]
    M = atom_xyz.shape[0]

    xaug = jnp.zeros((N_PAD, 8), jnp.float32)
    xaug = xaug.at[:N, :3].set(xyz)
    xaug = xaug.at[:, 3].set(1.0)
    xaug = xaug.at[:N, 4].set((xyz * xyz).sum(-1))
    a_p = jnp.full((M_PAD, 3), 1e17, jnp.float32).at[:M].set(atom_xyz)
    aaug = jnp.zeros((8, M_PAD), jnp.float32)
    aaug = aaug.at[:3].set(-2.0 * a_p.T)
    aaug = aaug.at[3].set((a_p * a_p).sum(-1))
    aaug = aaug.at[4].set(1.0)
    types_p = jnp.zeros((M_PAD, D), jnp.float32).at[:M].set(atom_types)

    table = _atom_mlp(types_p, Wt1, bt1, Wt2, bt2, Wt3, bt3)
    idx, dist = _topk(xaug, aaug)

    flat_idx = idx[:N].reshape(-1)
    flat_idx_p = jnp.zeros((B_PAD,), jnp.int32).at[:N * K].set(flat_idx)
    g = _sc_gather(table, flat_idx_p)

    gf = g[:N * K].reshape(N, K * D)
    gf = jnp.zeros((N_PAD, K * D), jnp.float32).at[:N].set(gf)
    d2p = jnp.ones((N_PAD, K), jnp.float32).at[:N].set(dist[:N])

    eye = jnp.eye(K, dtype=jnp.float32)
    w1bd = jnp.kron(eye, Wc1[:D, :])
    rw = jnp.kron(eye, Wc1[D:D + 1, :])
    w2bd = jnp.kron(eye, Wc2)
    b1t = jnp.tile(bc1, K).reshape(1, K * D)
    b2t = jnp.tile(bc2, K).reshape(1, K * D)
    g1t = jnp.tile(g1, K).reshape(1, K * D)
    be1t = jnp.tile(beta1, K).reshape(1, K * D)
    g2t = jnp.tile(g2, K).reshape(1, K * D)
    be2t = jnp.tile(beta2, K).reshape(1, K * D)

    h1, s1 = _c1(N, gf, d2p, w1bd, rw, b1t)
    h2, fx1, s2 = _c2(N, h1, s1, g1t, be1t, w2bd, b2t)
    out = _c3(N, h2, s2, g2t, be2t, fx1, Wc3[:D, :], Wc3[D:, :],
              bc3.reshape(1, D))
    return out[:N]


# speedup vs baseline: 1.1072x; 1.1072x over previous
import functools

import jax
import jax.numpy as jnp
from jax import lax
from jax.experimental import pallas as pl
from jax.experimental.pallas import tpu as pltpu
from jax.experimental.pallas import tpu_sc as plsc

K = 16
D = 16
BQ = 256
M_PAD = 8192
N_PAD = 10240
BQ2 = 512
NW = 32
B_PAD = 163840


def _leaky(x):
    return jnp.where(x >= 0, x, 0.2 * x)


def _atom_mlp_body(t_ref, w1_ref, b1_ref, w2_ref, b2_ref, w3_ref, b3_ref,
                   out_ref):
    x = t_ref[...]
    x = _leaky(jnp.dot(x, w1_ref[...],
                       preferred_element_type=jnp.float32) + b1_ref[...])
    x = _leaky(jnp.dot(x, w2_ref[...],
                       preferred_element_type=jnp.float32) + b2_ref[...])
    x = _leaky(jnp.dot(x, w3_ref[...],
                       preferred_element_type=jnp.float32) + b3_ref[...])
    out_ref[...] = x


def _atom_mlp(types_p, Wt1, bt1, Wt2, bt2, Wt3, bt3):
    return pl.pallas_call(
        _atom_mlp_body,
        out_shape=jax.ShapeDtypeStruct((M_PAD, D), jnp.float32),
    )(types_p, Wt1, bt1.reshape(1, D), Wt2, bt2.reshape(1, D),
      Wt3, bt3.reshape(1, D))


_NVJ = M_PAD // 128
_BIG = 3.3e38
_BIGI = 1 << 30


_DEPTH = 5


def _topk_body(xaug_ref, aaug_ref, idx_ref, dist_ref, w_ref):
    lane = lax.broadcasted_iota(jnp.int32, (1, 128), 1)
    x0 = xaug_ref[:, 0:1]
    x1 = xaug_ref[:, 1:2]
    x2 = xaug_ref[:, 2:3]
    xn = xaug_ref[:, 4:5]

    def d2sl(lo, hi):
        a0 = aaug_ref[0:1, lo:hi]
        a1 = aaug_ref[1:2, lo:hi]
        a2 = aaug_ref[2:3, lo:hi]
        an = aaug_ref[3:4, lo:hi]
        return (x0 * a0 + x1 * a1) + (x2 * a2 + (an + xn))

    sv = [jnp.full((BQ, 128), _BIG, jnp.float32) for _ in range(_DEPTH)]
    si = [jnp.full((BQ, 128), _BIGI, jnp.int32) for _ in range(_DEPTH)]
    for vj in range(_NVJ):
        xv = d2sl(128 * vj, 128 * (vj + 1))
        absv = lane + 128 * vj
        c = [xv < s for s in sv]
        for j in range(_DEPTH - 1, 0, -1):
            ins = jnp.where(c[j], xv, sv[j])
            sv[j] = jnp.where(c[j - 1], sv[j - 1], ins)
            insi = jnp.where(c[j], absv, si[j])
            si[j] = jnp.where(c[j - 1], si[j - 1], insi)
        sv[0] = jnp.where(c[0], xv, sv[0])
        si[0] = jnp.where(c[0], absv, si[0])

    vals, idxs = [], []
    fq = jnp.zeros((BQ, 1), jnp.bool_)
    for k in range(K):
        mv = jnp.min(sv[0], axis=1, keepdims=True)
        am = jnp.min(jnp.where(sv[0] <= mv, si[0], _BIGI), axis=1,
                     keepdims=True)
        vals.append(mv)
        idxs.append(am)
        m = lane == (am & 127)
        fq = fq | jnp.any(m & (sv[1] >= _BIG), axis=1, keepdims=True)
        for j in range(_DEPTH - 1):
            sv[j] = jnp.where(m, sv[j + 1], sv[j])
            si[j] = jnp.where(m, si[j + 1], si[j])
        sv[-1] = jnp.where(m, _BIG, sv[-1])
        si[-1] = jnp.where(m, _BIGI, si[-1])
    idx_ref[...] = jnp.concatenate(idxs, axis=1)
    dist_ref[...] = jnp.concatenate(vals, axis=1)

    @pl.when(jnp.any(fq))
    def _():
        w_ref[...] = d2sl(0, M_PAD)
        ii = lax.broadcasted_iota(jnp.int32, (BQ, M_PAD), 1)
        vals2, idxs2 = [], []
        for k in range(K):
            w = w_ref[...]
            mv = jnp.min(w, axis=1, keepdims=True)
            am = jnp.min(jnp.where(w <= mv, ii, _BIGI), axis=1,
                         keepdims=True)
            vals2.append(mv)
            idxs2.append(am)
            w_ref[...] = jnp.where(ii == am, _BIG, w)
        idx_ref[...] = jnp.where(fq, jnp.concatenate(idxs2, axis=1),
                                 idx_ref[...])
        dist_ref[...] = jnp.where(fq, jnp.concatenate(vals2, axis=1),
                                  dist_ref[...])


def _topk(xyz_p, axyz_p):
    return pl.pallas_call(
        _topk_body,
        grid=(N_PAD // BQ,),
        in_specs=[
            pl.BlockSpec((BQ, 8), lambda i: (i, 0)),
            pl.BlockSpec((8, M_PAD), lambda i: (0, 0)),
        ],
        out_specs=[
            pl.BlockSpec((BQ, K), lambda i: (i, 0)),
            pl.BlockSpec((BQ, K), lambda i: (i, 0)),
        ],
        out_shape=[
            jax.ShapeDtypeStruct((N_PAD, K), jnp.int32),
            jax.ShapeDtypeStruct((N_PAD, K), jnp.float32),
        ],
        scratch_shapes=[pltpu.VMEM((BQ, M_PAD), jnp.float32)],
        compiler_params=pltpu.CompilerParams(
            dimension_semantics=("arbitrary",)),
    )(xyz_p, axyz_p)


def _sc_gather_body(table_hbm, idx_hbm, out_hbm, idx_v, rows_v, sem):
    wid = lax.axis_index("s") * 2 + lax.axis_index("c")
    rows_per_w = B_PAD // NW
    base = wid * rows_per_w
    pltpu.sync_copy(idx_hbm.at[pl.ds(base, rows_per_w)], idx_v)
    pltpu.async_copy(table_hbm.at[idx_v], rows_v, sem).wait()
    pltpu.sync_copy(rows_v, out_hbm.at[pl.ds(base, rows_per_w)])


def _sc_gather(table, flat_idx_p):
    rows_per_w = B_PAD // NW
    mesh = plsc.VectorSubcoreMesh(core_axis_name="c", subcore_axis_name="s")
    f = pl.kernel(
        _sc_gather_body,
        out_type=jax.ShapeDtypeStruct((B_PAD, D), jnp.float32),
        mesh=mesh,
        scratch_types=[
            pltpu.VMEM((rows_per_w,), jnp.int32),
            pltpu.VMEM((rows_per_w, D), jnp.float32),
            pltpu.SemaphoreType.DMA,
        ],
        compiler_params=pltpu.CompilerParams(use_tc_tiling_on_sc=False),
    )
    return f(table, flat_idx_p)


def _c1_body(n_valid, g_ref, d_ref, w1bd_ref, rw_ref, b1t_ref,
             h1_ref, s_ref):
    i = pl.program_id(0)
    x = jnp.dot(g_ref[...], w1bd_ref[...], preferred_element_type=jnp.float32)
    inv = 1.0 / d_ref[...]
    x = x + jnp.dot(inv, rw_ref[...], preferred_element_type=jnp.float32)
    h = _leaky(x + b1t_ref[...])
    h1_ref[...] = h
    rows = i * BQ2 + lax.broadcasted_iota(jnp.int32, (BQ2, 1), 0)
    msk = rows < n_valid
    hm = jnp.where(msk, h, 0.0)
    hm2 = jnp.where(msk, h * h, 0.0)

    @pl.when(i == 0)
    def _():
        s_ref[...] = jnp.zeros_like(s_ref)

    s_ref[0:1, :] += jnp.sum(hm, axis=0, keepdims=True)
    s_ref[1:2, :] += jnp.sum(hm2, axis=0, keepdims=True)


def _bn_consts(s_ref, n_total, gt_ref, bt_ref):
    s = s_ref[...]
    tot = None
    tot2 = None
    for k in range(K):
        sl = s[0:1, 16 * k:16 * (k + 1)]
        sl2 = s[1:2, 16 * k:16 * (k + 1)]
        tot = sl if tot is None else tot + sl
        tot2 = sl2 if tot2 is None else tot2 + sl2
    mean = tot / n_total
    var = tot2 / n_total - mean * mean
    den = jnp.sqrt(var + 1e-5)
    mt = jnp.concatenate([mean] * K, axis=1)
    dent = jnp.concatenate([den] * K, axis=1)
    return mt, dent, gt_ref[...], bt_ref[...]


def _c2_body(n_valid, h1_ref, s1_ref, g1t_ref, be1t_ref, w2bd_ref, b2t_ref,
             h2_ref, fx1_ref, s_ref):
    i = pl.program_id(0)
    mt, dent, gt, bt = _bn_consts(s1_ref, jnp.float32(n_valid * K),
                                  g1t_ref, be1t_ref)
    u = (h1_ref[...] - mt) / dent * gt + bt
    fx1 = None
    for k in range(K):
        sl = u[:, 16 * k:16 * (k + 1)]
        fx1 = sl if fx1 is None else fx1 + sl
    fx1_ref[...] = fx1
    h = _leaky(jnp.dot(u, w2bd_ref[...],
                       preferred_element_type=jnp.float32) + b2t_ref[...])
    h2_ref[...] = h
    rows = i * BQ2 + lax.broadcasted_iota(jnp.int32, (BQ2, 1), 0)
    msk = rows < n_valid
    hm = jnp.where(msk, h, 0.0)
    hm2 = jnp.where(msk, h * h, 0.0)

    @pl.when(i == 0)
    def _():
        s_ref[...] = jnp.zeros_like(s_ref)

    s_ref[0:1, :] += jnp.sum(hm, axis=0, keepdims=True)
    s_ref[1:2, :] += jnp.sum(hm2, axis=0, keepdims=True)


def _c3_body(n_valid, h2_ref, s2_ref, g2t_ref, be2t_ref, fx1_ref,
             w3a_ref, w3b_ref, b3_ref, out_ref):
    mt, dent, gt, bt = _bn_consts(s2_ref, jnp.float32(n_valid * K),
                                  g2t_ref, be2t_ref)
    v = (h2_ref[...] - mt) / dent * gt + bt
    fx2 = None
    for k in range(K):
        sl = v[:, 16 * k:16 * (k + 1)]
        fx2 = sl if fx2 is None else fx2 + sl
    out_ref[...] = (jnp.dot(fx1_ref[...], w3a_ref[...],
                            preferred_element_type=jnp.float32)
                    + jnp.dot(fx2, w3b_ref[...],
                              preferred_element_type=jnp.float32)
                    + b3_ref[...])


def _whole(shape):
    return pl.BlockSpec(shape, lambda i: tuple(0 for _ in shape))


def _rowblk(w):
    return pl.BlockSpec((BQ2, w), lambda i: (i, 0))


def _c1(n_valid, g2, d2p, w1bd, rw, b1t):
    return pl.pallas_call(
        functools.partial(_c1_body, n_valid),
        grid=(N_PAD // BQ2,),
        in_specs=[_rowblk(K * D), _rowblk(K), _whole((K * D, K * D)),
                  _whole((K, K * D)), _whole((1, K * D))],
        out_specs=[_rowblk(K * D), _whole((8, K * D))],
        out_shape=[
            jax.ShapeDtypeStruct((N_PAD, K * D), jnp.float32),
            jax.ShapeDtypeStruct((8, K * D), jnp.float32),
        ],
        compiler_params=pltpu.CompilerParams(
            dimension_semantics=("arbitrary",)),
    )(g2, d2p, w1bd, rw, b1t)


def _c2(n_valid, h1, s1, g1t, be1t, w2bd, b2t):
    return pl.pallas_call(
        functools.partial(_c2_body, n_valid),
        grid=(N_PAD // BQ2,),
        in_specs=[_rowblk(K * D), _whole((8, K * D)), _whole((1, K * D)),
                  _whole((1, K * D)), _whole((K * D, K * D)),
                  _whole((1, K * D))],
        out_specs=[_rowblk(K * D), _rowblk(D), _whole((8, K * D))],
        out_shape=[
            jax.ShapeDtypeStruct((N_PAD, K * D), jnp.float32),
            jax.ShapeDtypeStruct((N_PAD, D), jnp.float32),
            jax.ShapeDtypeStruct((8, K * D), jnp.float32),
        ],
        compiler_params=pltpu.CompilerParams(
            dimension_semantics=("arbitrary",)),
    )(h1, s1, g1t, be1t, w2bd, b2t)


def _c3(n_valid, h2, s2, g2t, be2t, fx1, w3a, w3b, b3):
    return pl.pallas_call(
        functools.partial(_c3_body, n_valid),
        grid=(N_PAD // BQ2,),
        in_specs=[_rowblk(K * D), _whole((8, K * D)), _whole((1, K * D)),
                  _whole((1, K * D)), _rowblk(D), _whole((D, D)),
                  _whole((D, D)), _whole((1, D))],
        out_specs=_rowblk(D),
        out_shape=jax.ShapeDtypeStruct((N_PAD, D), jnp.float32),
        compiler_params=pltpu.CompilerParams(
            dimension_semantics=("arbitrary",)),
    )(h2, s2, g2t, be2t, fx1, w3a, w3b, b3)


def kernel(xyz, atom_xyz, atom_types, Wt1, bt1, Wt2, bt2, Wt3, bt3,
           Wc1, bc1, Wc2, bc2, Wc3, bc3, g1, beta1, g2, beta2):
    N = xyz.shape[0]
    M = atom_xyz.shape[0]

    xaug = jnp.zeros((N_PAD, 8), jnp.float32)
    xaug = xaug.at[:N, :3].set(xyz)
    xaug = xaug.at[:, 3].set(1.0)
    xaug = xaug.at[:N, 4].set((xyz * xyz).sum(-1))
    a_p = jnp.full((M_PAD, 3), 1e17, jnp.float32).at[:M].set(atom_xyz)
    aaug = jnp.zeros((8, M_PAD), jnp.float32)
    aaug = aaug.at[:3].set(-2.0 * a_p.T)
    aaug = aaug.at[3].set((a_p * a_p).sum(-1))
    aaug = aaug.at[4].set(1.0)
    types_p = jnp.zeros((M_PAD, D), jnp.float32).at[:M].set(atom_types)

    table = _atom_mlp(types_p, Wt1, bt1, Wt2, bt2, Wt3, bt3)
    idx, dist = _topk(xaug, aaug)

    g = _sc_gather(table, idx.reshape(-1))
    gf = g.reshape(N_PAD, K * D)
    d2p = dist

    eye = jnp.eye(K, dtype=jnp.float32)
    w1bd = jnp.kron(eye, Wc1[:D, :])
    rw = jnp.kron(eye, Wc1[D:D + 1, :])
    w2bd = jnp.kron(eye, Wc2)
    b1t = jnp.tile(bc1, K).reshape(1, K * D)
    b2t = jnp.tile(bc2, K).reshape(1, K * D)
    g1t = jnp.tile(g1, K).reshape(1, K * D)
    be1t = jnp.tile(beta1, K).reshape(1, K * D)
    g2t = jnp.tile(g2, K).reshape(1, K * D)
    be2t = jnp.tile(beta2, K).reshape(1, K * D)

    h1, s1 = _c1(N, gf, d2p, w1bd, rw, b1t)
    h2, fx1, s2 = _c2(N, h1, s1, g1t, be1t, w2bd, b2t)
    out = _c3(N, h2, s2, g2t, be2t, fx1, Wc3[:D, :], Wc3[D:, :],
              bc3.reshape(1, D))
    return out[:N]


# speedup vs baseline: 2.3799x; 2.1494x over previous
import functools

import jax
import jax.numpy as jnp
from jax import lax
from jax.experimental import pallas as pl
from jax.experimental.pallas import tpu as pltpu
from jax.experimental.pallas import tpu_sc as plsc

K = 16
D = 16
BQ = 256
M_PAD = 8192
N_PAD = 10240
BQ2 = 512
NW = 32
B_PAD = 163840


def _leaky(x):
    return jnp.where(x >= 0, x, 0.2 * x)


def _atom_mlp_body(t_ref, w1_ref, b1_ref, w2_ref, b2_ref, w3_ref, b3_ref,
                   out_ref):
    x = t_ref[...]
    x = _leaky(jnp.dot(x, w1_ref[...],
                       preferred_element_type=jnp.float32) + b1_ref[...])
    x = _leaky(jnp.dot(x, w2_ref[...],
                       preferred_element_type=jnp.float32) + b2_ref[...])
    x = _leaky(jnp.dot(x, w3_ref[...],
                       preferred_element_type=jnp.float32) + b3_ref[...])
    out_ref[...] = x


def _atom_mlp(types_p, Wt1, bt1, Wt2, bt2, Wt3, bt3):
    return pl.pallas_call(
        _atom_mlp_body,
        out_shape=jax.ShapeDtypeStruct((M_PAD, D), jnp.float32),
    )(types_p, Wt1, bt1.reshape(1, D), Wt2, bt2.reshape(1, D),
      Wt3, bt3.reshape(1, D))


_NVJ = M_PAD // 128
_BIG = 3.3e38
_BIGI = 1 << 30


_DEPTH = 5


_NIDX = 4
_MAGIC = 1 << 20


def _topk_body(xaug_ref, aaug_ref, idx_ref, dist_ref, w_ref):
    lane = lax.broadcasted_iota(jnp.int32, (1, 128), 1)
    w_ref[...] = jnp.dot(xaug_ref[...], aaug_ref[...],
                         preferred_element_type=jnp.float32)

    sv = [jnp.full((BQ, 128), _BIG, jnp.float32) for _ in range(_DEPTH)]
    si = [jnp.full((BQ, 128), _BIGI, jnp.int32) for _ in range(_NIDX)]
    for vj in range(_NVJ):
        xv = w_ref[:, 128 * vj:128 * (vj + 1)]
        absv = lane + 128 * vj
        c = [xv < s for s in sv]
        for j in range(_DEPTH - 1, 0, -1):
            ins = jnp.where(c[j], xv, sv[j])
            sv[j] = jnp.where(c[j - 1], sv[j - 1], ins)
            if j < _NIDX:
                insi = jnp.where(c[j], absv, si[j])
                si[j] = jnp.where(c[j - 1], si[j - 1], insi)
        sv[0] = jnp.where(c[0], xv, sv[0])
        si[0] = jnp.where(c[0], absv, si[0])

    vals, idxs = [], []
    fq = jnp.zeros((BQ, 1), jnp.bool_)
    for k in range(K):
        mv = jnp.min(sv[0], axis=1, keepdims=True)
        am = jnp.min(jnp.where(sv[0] <= mv, si[0], _BIGI), axis=1,
                     keepdims=True)
        vals.append(mv)
        idxs.append(am)
        fq = fq | (am >= _MAGIC)
        m = lane == (am & 127)
        for j in range(_DEPTH - 1):
            sv[j] = jnp.where(m, sv[j + 1], sv[j])
        sv[-1] = jnp.where(m, _BIG, sv[-1])
        for j in range(_NIDX - 1):
            si[j] = jnp.where(m, si[j + 1], si[j])
        si[-1] = jnp.where(m, _MAGIC, si[-1])
    idx_ref[...] = jnp.concatenate(idxs, axis=1)
    dist_ref[...] = jnp.concatenate(vals, axis=1)

    @pl.when(jnp.any(fq))
    def _():
        ii = lax.broadcasted_iota(jnp.int32, (BQ, M_PAD), 1)
        vals2, idxs2 = [], []
        for k in range(K):
            w = w_ref[...]
            mv = jnp.min(w, axis=1, keepdims=True)
            am = jnp.min(jnp.where(w <= mv, ii, _BIGI), axis=1,
                         keepdims=True)
            vals2.append(mv)
            idxs2.append(am)
            w_ref[...] = jnp.where(ii == am, _BIG, w)
        idx_ref[...] = jnp.where(fq, jnp.concatenate(idxs2, axis=1),
                                 idx_ref[...])
        dist_ref[...] = jnp.where(fq, jnp.concatenate(vals2, axis=1),
                                  dist_ref[...])


def _topk(xyz_p, axyz_p):
    return pl.pallas_call(
        _topk_body,
        grid=(N_PAD // BQ,),
        in_specs=[
            pl.BlockSpec((BQ, 8), lambda i: (i, 0)),
            pl.BlockSpec((8, M_PAD), lambda i: (0, 0)),
        ],
        out_specs=[
            pl.BlockSpec((BQ, K), lambda i: (i, 0)),
            pl.BlockSpec((BQ, K), lambda i: (i, 0)),
        ],
        out_shape=[
            jax.ShapeDtypeStruct((N_PAD, K), jnp.int32),
            jax.ShapeDtypeStruct((N_PAD, K), jnp.float32),
        ],
        scratch_shapes=[pltpu.VMEM((BQ, M_PAD), jnp.float32)],
        compiler_params=pltpu.CompilerParams(
            dimension_semantics=("arbitrary",)),
    )(xyz_p, axyz_p)


def _sc_gather_body(table_hbm, idx_hbm, out_hbm, idx_v, rows_v, sem):
    wid = lax.axis_index("s") * 2 + lax.axis_index("c")
    rows_per_w = B_PAD // NW
    base = wid * rows_per_w
    pltpu.sync_copy(idx_hbm.at[pl.ds(base, rows_per_w)], idx_v)
    pltpu.async_copy(table_hbm.at[idx_v], rows_v, sem).wait()
    pltpu.sync_copy(rows_v, out_hbm.at[pl.ds(base, rows_per_w)])


def _sc_gather(table, flat_idx_p):
    rows_per_w = B_PAD // NW
    mesh = plsc.VectorSubcoreMesh(core_axis_name="c", subcore_axis_name="s")
    f = pl.kernel(
        _sc_gather_body,
        out_type=jax.ShapeDtypeStruct((B_PAD, D), jnp.float32),
        mesh=mesh,
        scratch_types=[
            pltpu.VMEM((rows_per_w,), jnp.int32),
            pltpu.VMEM((rows_per_w, D), jnp.float32),
            pltpu.SemaphoreType.DMA,
        ],
        compiler_params=pltpu.CompilerParams(use_tc_tiling_on_sc=False),
    )
    return f(table, flat_idx_p)


def _c1_body(n_valid, g_ref, d_ref, w1bd_ref, rw_ref, b1t_ref,
             h1_ref, s_ref):
    i = pl.program_id(0)
    x = jnp.dot(g_ref[...], w1bd_ref[...], preferred_element_type=jnp.float32)
    inv = 1.0 / d_ref[...]
    x = x + jnp.dot(inv, rw_ref[...], preferred_element_type=jnp.float32)
    h = _leaky(x + b1t_ref[...])
    h1_ref[...] = h
    rows = i * BQ2 + lax.broadcasted_iota(jnp.int32, (BQ2, 1), 0)
    msk = rows < n_valid
    hm = jnp.where(msk, h, 0.0)
    hm2 = jnp.where(msk, h * h, 0.0)

    @pl.when(i == 0)
    def _():
        s_ref[...] = jnp.zeros_like(s_ref)

    s_ref[0:1, :] += jnp.sum(hm, axis=0, keepdims=True)
    s_ref[1:2, :] += jnp.sum(hm2, axis=0, keepdims=True)


def _bn_consts(s_ref, n_total, gt_ref, bt_ref):
    s = s_ref[...]
    tot = None
    tot2 = None
    for k in range(K):
        sl = s[0:1, 16 * k:16 * (k + 1)]
        sl2 = s[1:2, 16 * k:16 * (k + 1)]
        tot = sl if tot is None else tot + sl
        tot2 = sl2 if tot2 is None else tot2 + sl2
    mean = tot / n_total
    var = tot2 / n_total - mean * mean
    den = jnp.sqrt(var + 1e-5)
    mt = jnp.concatenate([mean] * K, axis=1)
    dent = jnp.concatenate([den] * K, axis=1)
    return mt, dent, gt_ref[...], bt_ref[...]


def _c2_body(n_valid, h1_ref, s1_ref, g1t_ref, be1t_ref, w2bd_ref, b2t_ref,
             h2_ref, fx1_ref, s_ref):
    i = pl.program_id(0)
    mt, dent, gt, bt = _bn_consts(s1_ref, jnp.float32(n_valid * K),
                                  g1t_ref, be1t_ref)
    u = (h1_ref[...] - mt) / dent * gt + bt
    fx1 = None
    for k in range(K):
        sl = u[:, 16 * k:16 * (k + 1)]
        fx1 = sl if fx1 is None else fx1 + sl
    fx1_ref[...] = fx1
    h = _leaky(jnp.dot(u, w2bd_ref[...],
                       preferred_element_type=jnp.float32) + b2t_ref[...])
    h2_ref[...] = h
    rows = i * BQ2 + lax.broadcasted_iota(jnp.int32, (BQ2, 1), 0)
    msk = rows < n_valid
    hm = jnp.where(msk, h, 0.0)
    hm2 = jnp.where(msk, h * h, 0.0)

    @pl.when(i == 0)
    def _():
        s_ref[...] = jnp.zeros_like(s_ref)

    s_ref[0:1, :] += jnp.sum(hm, axis=0, keepdims=True)
    s_ref[1:2, :] += jnp.sum(hm2, axis=0, keepdims=True)


def _c3_body(n_valid, h2_ref, s2_ref, g2t_ref, be2t_ref, fx1_ref,
             w3a_ref, w3b_ref, b3_ref, out_ref):
    mt, dent, gt, bt = _bn_consts(s2_ref, jnp.float32(n_valid * K),
                                  g2t_ref, be2t_ref)
    v = (h2_ref[...] - mt) / dent * gt + bt
    fx2 = None
    for k in range(K):
        sl = v[:, 16 * k:16 * (k + 1)]
        fx2 = sl if fx2 is None else fx2 + sl
    out_ref[...] = (jnp.dot(fx1_ref[...], w3a_ref[...],
                            preferred_element_type=jnp.float32)
                    + jnp.dot(fx2, w3b_ref[...],
                              preferred_element_type=jnp.float32)
                    + b3_ref[...])


def _whole(shape):
    return pl.BlockSpec(shape, lambda i: tuple(0 for _ in shape))


def _rowblk(w):
    return pl.BlockSpec((BQ2, w), lambda i: (i, 0))


def _c1(n_valid, g2, d2p, w1bd, rw, b1t):
    return pl.pallas_call(
        functools.partial(_c1_body, n_valid),
        grid=(N_PAD // BQ2,),
        in_specs=[_rowblk(K * D), _rowblk(K), _whole((K * D, K * D)),
                  _whole((K, K * D)), _whole((1, K * D))],
        out_specs=[_rowblk(K * D), _whole((8, K * D))],
        out_shape=[
            jax.ShapeDtypeStruct((N_PAD, K * D), jnp.float32),
            jax.ShapeDtypeStruct((8, K * D), jnp.float32),
        ],
        compiler_params=pltpu.CompilerParams(
            dimension_semantics=("arbitrary",)),
    )(g2, d2p, w1bd, rw, b1t)


def _c2(n_valid, h1, s1, g1t, be1t, w2bd, b2t):
    return pl.pallas_call(
        functools.partial(_c2_body, n_valid),
        grid=(N_PAD // BQ2,),
        in_specs=[_rowblk(K * D), _whole((8, K * D)), _whole((1, K * D)),
                  _whole((1, K * D)), _whole((K * D, K * D)),
                  _whole((1, K * D))],
        out_specs=[_rowblk(K * D), _rowblk(D), _whole((8, K * D))],
        out_shape=[
            jax.ShapeDtypeStruct((N_PAD, K * D), jnp.float32),
            jax.ShapeDtypeStruct((N_PAD, D), jnp.float32),
            jax.ShapeDtypeStruct((8, K * D), jnp.float32),
        ],
        compiler_params=pltpu.CompilerParams(
            dimension_semantics=("arbitrary",)),
    )(h1, s1, g1t, be1t, w2bd, b2t)


def _c3(n_valid, h2, s2, g2t, be2t, fx1, w3a, w3b, b3):
    return pl.pallas_call(
        functools.partial(_c3_body, n_valid),
        grid=(N_PAD // BQ2,),
        in_specs=[_rowblk(K * D), _whole((8, K * D)), _whole((1, K * D)),
                  _whole((1, K * D)), _rowblk(D), _whole((D, D)),
                  _whole((D, D)), _whole((1, D))],
        out_specs=_rowblk(D),
        out_shape=jax.ShapeDtypeStruct((N_PAD, D), jnp.float32),
        compiler_params=pltpu.CompilerParams(
            dimension_semantics=("arbitrary",)),
    )(h2, s2, g2t, be2t, fx1, w3a, w3b, b3)


def kernel(xyz, atom_xyz, atom_types, Wt1, bt1, Wt2, bt2, Wt3, bt3,
           Wc1, bc1, Wc2, bc2, Wc3, bc3, g1, beta1, g2, beta2):
    N = xyz.shape[0]
    M = atom_xyz.shape[0]

    xaug = jnp.zeros((N_PAD, 8), jnp.float32)
    xaug = xaug.at[:N, :3].set(xyz)
    xaug = xaug.at[:, 3].set(1.0)
    xaug = xaug.at[:N, 4].set((xyz * xyz).sum(-1))
    a_p = jnp.full((M_PAD, 3), 1e17, jnp.float32).at[:M].set(atom_xyz)
    aaug = jnp.zeros((8, M_PAD), jnp.float32)
    aaug = aaug.at[:3].set(-2.0 * a_p.T)
    aaug = aaug.at[3].set((a_p * a_p).sum(-1))
    aaug = aaug.at[4].set(1.0)
    types_p = jnp.zeros((M_PAD, D), jnp.float32).at[:M].set(atom_types)

    table = _atom_mlp(types_p, Wt1, bt1, Wt2, bt2, Wt3, bt3)
    idx, dist = _topk(xaug, aaug)

    g = _sc_gather(table, idx.reshape(-1))
    gf = g.reshape(N_PAD, K * D)
    d2p = dist

    eye = jnp.eye(K, dtype=jnp.float32)
    w1bd = jnp.kron(eye, Wc1[:D, :])
    rw = jnp.kron(eye, Wc1[D:D + 1, :])
    w2bd = jnp.kron(eye, Wc2)
    b1t = jnp.tile(bc1, K).reshape(1, K * D)
    b2t = jnp.tile(bc2, K).reshape(1, K * D)
    g1t = jnp.tile(g1, K).reshape(1, K * D)
    be1t = jnp.tile(beta1, K).reshape(1, K * D)
    g2t = jnp.tile(g2, K).reshape(1, K * D)
    be2t = jnp.tile(beta2, K).reshape(1, K * D)

    h1, s1 = _c1(N, gf, d2p, w1bd, rw, b1t)
    h2, fx1, s2 = _c2(N, h1, s1, g1t, be1t, w2bd, b2t)
    out = _c3(N, h2, s2, g2t, be2t, fx1, Wc3[:D, :], Wc3[D:, :],
              bc3.reshape(1, D))
    return out[:N]
